# baseline (speedup 1.0000x reference)
import functools

import jax
import jax.numpy as jnp
from jax import lax
from jax.experimental import pallas as pl
from jax.experimental.pallas import tpu as pltpu
from jax.experimental.pallas import tpu_sc as plsc

BN = 1000


def _premult_body(h_ref, w_ref, hrw_ref):
    R = w_ref.shape[0]
    h = h_ref[...]
    for r in range(R):
        hrw_ref[r] = h * w_ref[r]


def _make_premult(N, C, R):
    nb = N // BN
    return pl.pallas_call(
        _premult_body,
        grid=(nb,),
        in_specs=[
            pl.BlockSpec((BN, C), lambda i: (i, 0)),
            pl.BlockSpec((R, C), lambda i: (0, 0)),
        ],
        out_specs=pl.BlockSpec((R, BN, C), lambda i: (0, i, 0)),
        out_shape=jax.ShapeDtypeStruct((R, N, C), jnp.float32),
    )


def _mlp_body(eps_ref, h_ref, agg_ref, w1_ref, b1_ref, w2_ref, b2_ref,
              *rest):
    ht = (1.0 + eps_ref[0, 0]) * h_ref[...] + agg_ref[0] + agg_ref[1]
    hmid = jnp.maximum(
        jnp.dot(ht, w1_ref[...], preferred_element_type=jnp.float32)
        + b1_ref[...], 0.0)
    out = (jnp.dot(hmid, w2_ref[...], preferred_element_type=jnp.float32)
           + b2_ref[...])
    if len(rest) == 1:
        (out_ref,) = rest
        out_ref[...] = out
    else:
        wn_ref, out_ref, hrw_ref = rest
        out_ref[...] = out
        for r in range(wn_ref.shape[0]):
            hrw_ref[r] = out * wn_ref[r]


def _make_mlp(N, C, R, fuse_premult):
    nb = N // BN
    in_specs = [
        pl.BlockSpec((1, 1), lambda i: (0, 0)),
        pl.BlockSpec((BN, C), lambda i: (i, 0)),
        pl.BlockSpec((2, BN, C), lambda i: (0, i, 0)),
        pl.BlockSpec((C, C), lambda i: (0, 0)),
        pl.BlockSpec((1, C), lambda i: (0, 0)),
        pl.BlockSpec((C, C), lambda i: (0, 0)),
        pl.BlockSpec((1, C), lambda i: (0, 0)),
    ]
    out_specs = pl.BlockSpec((BN, C), lambda i: (i, 0))
    out_shape = jax.ShapeDtypeStruct((N, C), jnp.float32)
    if fuse_premult:
        in_specs.append(pl.BlockSpec((R, C), lambda i: (0, 0)))
        out_specs = [out_specs, pl.BlockSpec((R, BN, C), lambda i: (0, i, 0))]
        out_shape = [out_shape, jax.ShapeDtypeStruct((R, N, C), jnp.float32)]
    return pl.pallas_call(
        _mlp_body,
        grid=(nb,),
        in_specs=in_specs,
        out_specs=out_specs,
        out_shape=out_shape,
    )


def _make_sc_agg(N, C, E):
    NC, NS = 2, 16
    NW = NC * NS
    W = 128
    NCH = E // W
    NF = NCH // NW
    NEXTRA = NCH - NF * NW
    assert (NF - 3) % 3 == 0 and NF >= 6
    RPS = (N // NS) // 8 * 8
    RLAST = N - (NS - 1) * RPS
    mesh = plsc.VectorSubcoreMesh(core_axis_name="c", subcore_axis_name="s")

    @functools.partial(
        pl.kernel,
        out_type=jax.ShapeDtypeStruct((NC, N, C), jnp.float32),
        mesh=mesh,
        scratch_types=[
            pltpu.VMEM((2, W), jnp.int32),
            pltpu.VMEM((2, W), jnp.int32),
            pltpu.VMEM((2, W), jnp.int32),
            pltpu.VMEM((1, W), jnp.int32),
            pltpu.VMEM((1, W), jnp.int32),
            pltpu.VMEM((1, W), jnp.int32),
            pltpu.VMEM((W, C), jnp.float32),
            pltpu.VMEM((W, C), jnp.float32),
            pltpu.VMEM((W, C), jnp.float32),
            pltpu.VMEM_SHARED((N, C), jnp.float32),
            pltpu.SemaphoreType.DMA,
            pltpu.SemaphoreType.DMA,
            pltpu.SemaphoreType.DMA,
            pltpu.SemaphoreType.DMA,
            pltpu.SemaphoreType.DMA,
            pltpu.SemaphoreType.DMA,
            pltpu.SemaphoreType.DMA,
            pltpu.SemaphoreType.DMA,
            pltpu.SemaphoreType.DMA,
            pltpu.SemaphoreType.DMA,
            pltpu.SemaphoreType.DMA,
            pltpu.SemaphoreType.DMA,
            pltpu.SemaphoreType.DMA,
        ],
    )
    def sc_agg(hrw_hbm, ei_hbm, et_hbm, zeros_hbm, out_hbm,
               ib0, ib1, ib2, eb0, eb1, eb2, rb0, rb1, rb2, acc_sh,
               gsem0, gsem1, gsem2, hsem0, hsem1, hsem2,
               isem0, isem1, isem2, esem0, esem1, esem2, zsem):
        cid = lax.axis_index("c")
        sid = lax.axis_index("s")
        wid = cid * NS + sid
        row_base = pl.multiple_of(sid * RPS, 8)
        last_base = (NS - 1) * RPS + RPS
        cbase = wid * NF

        ibs = (ib0, ib1, ib2)
        ebs = (eb0, eb1, eb2)
        rbs = (rb0, rb1, rb2)
        gsems = (gsem0, gsem1, gsem2)
        hsems = (hsem0, hsem1, hsem2)
        isems = (isem0, isem1, isem2)
        esems = (esem0, esem1, esem2)
        HW = W // 2

        pltpu.async_copy(zeros_hbm.at[pl.ds(0, RPS)],
                         acc_sh.at[pl.ds(row_base, RPS)], zsem)

        @pl.when(sid == NS - 1)
        def _():
            if RLAST != RPS:
                pltpu.async_copy(
                    zeros_hbm.at[pl.ds(RPS, RLAST - RPS)],
                    acc_sh.at[pl.ds(last_base, RLAST - RPS)], zsem)

        def load_idx(s, j):
            off = (cbase + s) * W
            pltpu.async_copy(ei_hbm.at[:, pl.ds(off, W)], ibs[j], isems[j])
            pltpu.async_copy(et_hbm.at[:, pl.ds(off, W)], ebs[j], esems[j])

        def wait_idx(s, j):
            off = (cbase + s) * W
            pltpu.make_async_copy(ei_hbm.at[:, pl.ds(off, W)], ibs[j],
                                  isems[j]).wait()
            pltpu.make_async_copy(et_hbm.at[:, pl.ds(off, W)], ebs[j],
                                  esems[j]).wait()
            for k in range(W // 16):
                sl = pl.ds(16 * k, 16)
                ibs[j][0, sl] = ebs[j][0, sl] * N + ibs[j][0, sl]

        def gather(j):
            pltpu.async_copy(hrw_hbm.at[ibs[j].at[0, pl.ds(0, HW)]],
                             rbs[j].at[pl.ds(0, HW)], gsems[j])
            pltpu.async_copy(hrw_hbm.at[ibs[j].at[0, pl.ds(HW, HW)]],
                             rbs[j].at[pl.ds(HW, HW)], hsems[j])

        def wait_gather(j):
            pltpu.make_async_copy(hrw_hbm.at[ibs[j].at[0, pl.ds(0, HW)]],
                                  rbs[j].at[pl.ds(0, HW)], gsems[j]).wait()
            pltpu.make_async_copy(hrw_hbm.at[ibs[j].at[0, pl.ds(HW, HW)]],
                                  rbs[j].at[pl.ds(HW, HW)], hsems[j]).wait()

        def scat(j):
            pltpu.sync_copy(rbs[j], acc_sh.at[ibs[j].at[1]], add=True)

        load_idx(0, 0)
        load_idx(1, 1)
        wait_idx(0, 0)
        wait_idx(1, 1)
        gather(0)
        gather(1)
        load_idx(2, 2)

        pltpu.make_async_copy(zeros_hbm.at[pl.ds(0, RPS)],
                              acc_sh.at[pl.ds(row_base, RPS)], zsem).wait()

        @pl.when(sid == NS - 1)
        def _():
            if RLAST != RPS:
                pltpu.make_async_copy(
                    zeros_hbm.at[pl.ds(RPS, RLAST - RPS)],
                    acc_sh.at[pl.ds(last_base, RLAST - RPS)], zsem).wait()

        plsc.subcore_barrier()

        @pl.loop(0, (NF - 3) // 3)
        def _(g):
            for j in range(3):
                s = g * 3 + j
                wait_gather(j)
                wait_idx(s + 2, (j + 2) % 3)
                gather((j + 2) % 3)
                scat(j)
                load_idx(s + 3, j)

        wait_gather(0)
        wait_idx(NF - 1, 2)
        gather(2)
        scat(0)

        xoff = (NW * NF + wid) * W

        @pl.when(wid < NEXTRA)
        def _():
            pltpu.async_copy(ei_hbm.at[:, pl.ds(xoff, W)], ib0, isem0)
            pltpu.async_copy(et_hbm.at[:, pl.ds(xoff, W)], eb0, esem0)

        wait_gather(1)
        scat(1)

        @pl.when(wid < NEXTRA)
        def _():
            pltpu.make_async_copy(ei_hbm.at[:, pl.ds(xoff, W)], ib0,
                                  isem0).wait()
            pltpu.make_async_copy(et_hbm.at[:, pl.ds(xoff, W)], eb0,
                                  esem0).wait()
            for k in range(W // 16):
                sl = pl.ds(16 * k, 16)
                ib0[0, sl] = eb0[0, sl] * N + ib0[0, sl]
            pltpu.async_copy(hrw_hbm.at[ib0.at[0]], rb0, gsem0)

        wait_gather(2)
        scat(2)

        @pl.when(wid < NEXTRA)
        def _():
            pltpu.make_async_copy(hrw_hbm.at[ib0.at[0]], rb0, gsem0).wait()
            pltpu.sync_copy(rb0, acc_sh.at[ib0.at[1]], add=True)

        plsc.subcore_barrier()

        @pl.when(sid != NS - 1)
        def _():
            pltpu.sync_copy(acc_sh.at[pl.ds(row_base, RPS)],
                            out_hbm.at[cid, pl.ds(row_base, RPS)])

        @pl.when(sid == NS - 1)
        def _():
            pltpu.sync_copy(acc_sh.at[pl.ds((NS - 1) * RPS, RLAST)],
                            out_hbm.at[cid, pl.ds((NS - 1) * RPS, RLAST)])

    return sc_agg


def kernel(x, edge_index, edge_type, num_edge_types,
           w0, eps0, m0w1, m0b1, m0w2, m0b2,
           w1, eps1, m1w1, m1b1, m1w2, m1b2):
    N, C = x.shape
    E = edge_type.shape[0]
    R = w0.shape[0]

    sc_agg = _make_sc_agg(N, C, E)
    mlp_fused = _make_mlp(N, C, R, fuse_premult=True)
    mlp_final = _make_mlp(N, C, R, fuse_premult=False)

    NS = 16
    rps = (N // NS) // 8 * 8
    rlast = N - (NS - 1) * rps
    zeros = jnp.zeros((rlast, C), jnp.float32)
    et2 = edge_type.reshape(1, E)
    hrw0 = _make_premult(N, C, R)(x, w0)
    part0 = sc_agg(hrw0.reshape(R * N, C), edge_index, et2, zeros)
    h1, hrw1 = mlp_fused(eps0.reshape(1, 1), x, part0,
                         m0w1, m0b1.reshape(1, C), m0w2, m0b2.reshape(1, C),
                         w1)
    part1 = sc_agg(hrw1.reshape(R * N, C), edge_index, et2, zeros)
    out = mlp_final(eps1.reshape(1, 1), h1, part1,
                    m1w1, m1b1.reshape(1, C), m1w2, m1b2.reshape(1, C))
    return out


# speedup vs baseline: 1.0199x; 1.0199x over previous
import functools

import jax
import jax.numpy as jnp
from jax import lax
from jax.experimental import pallas as pl
from jax.experimental.pallas import tpu as pltpu
from jax.experimental.pallas import tpu_sc as plsc

BN = 1000


def _premult_body(h_ref, w_ref, hrw_ref):
    R = w_ref.shape[0]
    h = h_ref[...]
    for r in range(R):
        hrw_ref[r] = h * w_ref[r]


def _make_premult(N, C, R):
    nb = N // BN
    return pl.pallas_call(
        _premult_body,
        grid=(nb,),
        in_specs=[
            pl.BlockSpec((BN, C), lambda i: (i, 0)),
            pl.BlockSpec((R, C), lambda i: (0, 0)),
        ],
        out_specs=pl.BlockSpec((R, BN, C), lambda i: (0, i, 0)),
        out_shape=jax.ShapeDtypeStruct((R, N, C), jnp.float32),
    )


def _mlp_body(eps_ref, h_ref, agg_ref, w1_ref, b1_ref, w2_ref, b2_ref,
              *rest):
    ht = (1.0 + eps_ref[0, 0]) * h_ref[...] + agg_ref[0] + agg_ref[1]
    hmid = jnp.maximum(
        jnp.dot(ht, w1_ref[...], preferred_element_type=jnp.float32)
        + b1_ref[...], 0.0)
    out = (jnp.dot(hmid, w2_ref[...], preferred_element_type=jnp.float32)
           + b2_ref[...])
    if len(rest) == 1:
        (out_ref,) = rest
        out_ref[...] = out
    else:
        wn_ref, out_ref, hrw_ref = rest
        out_ref[...] = out
        for r in range(wn_ref.shape[0]):
            hrw_ref[r] = out * wn_ref[r]


def _make_mlp(N, C, R, fuse_premult):
    nb = N // BN
    in_specs = [
        pl.BlockSpec((1, 1), lambda i: (0, 0)),
        pl.BlockSpec((BN, C), lambda i: (i, 0)),
        pl.BlockSpec((2, BN, C), lambda i: (0, i, 0)),
        pl.BlockSpec((C, C), lambda i: (0, 0)),
        pl.BlockSpec((1, C), lambda i: (0, 0)),
        pl.BlockSpec((C, C), lambda i: (0, 0)),
        pl.BlockSpec((1, C), lambda i: (0, 0)),
    ]
    out_specs = pl.BlockSpec((BN, C), lambda i: (i, 0))
    out_shape = jax.ShapeDtypeStruct((N, C), jnp.float32)
    if fuse_premult:
        in_specs.append(pl.BlockSpec((R, C), lambda i: (0, 0)))
        out_specs = [out_specs, pl.BlockSpec((R, BN, C), lambda i: (0, i, 0))]
        out_shape = [out_shape, jax.ShapeDtypeStruct((R, N, C), jnp.float32)]
    return pl.pallas_call(
        _mlp_body,
        grid=(nb,),
        in_specs=in_specs,
        out_specs=out_specs,
        out_shape=out_shape,
    )


def _make_sc_agg(N, C, E):
    NC, NS = 2, 16
    NW = NC * NS
    W = 128
    NCH = E // W
    NF = NCH // NW
    NEXTRA = NCH - NF * NW
    assert (NF - 3) % 3 == 0 and NF >= 6
    RPS = (N // NS) // 8 * 8
    RLAST = N - (NS - 1) * RPS
    mesh = plsc.VectorSubcoreMesh(core_axis_name="c", subcore_axis_name="s")

    @functools.partial(
        pl.kernel,
        out_type=jax.ShapeDtypeStruct((NC, N, C), jnp.float32),
        mesh=mesh,
        scratch_types=[
            pltpu.VMEM((2, W), jnp.int32),
            pltpu.VMEM((2, W), jnp.int32),
            pltpu.VMEM((2, W), jnp.int32),
            pltpu.VMEM((W,), jnp.int32),
            pltpu.VMEM((W,), jnp.int32),
            pltpu.VMEM((W,), jnp.int32),
            pltpu.VMEM((W, C), jnp.float32),
            pltpu.VMEM((W, C), jnp.float32),
            pltpu.VMEM((W, C), jnp.float32),
            pltpu.VMEM_SHARED((N, C), jnp.float32),
            pltpu.SemaphoreType.DMA,
            pltpu.SemaphoreType.DMA,
            pltpu.SemaphoreType.DMA,
            pltpu.SemaphoreType.DMA,
            pltpu.SemaphoreType.DMA,
            pltpu.SemaphoreType.DMA,
            pltpu.SemaphoreType.DMA,
            pltpu.SemaphoreType.DMA,
            pltpu.SemaphoreType.DMA,
            pltpu.SemaphoreType.DMA,
        ],
    )
    def sc_agg(hrw_hbm, ei_hbm, et_hbm, zeros_hbm, out_hbm,
               ib0, ib1, ib2, eb0, eb1, eb2, rb0, rb1, rb2, acc_sh,
               gsem0, gsem1, gsem2, isem0, isem1, isem2,
               esem0, esem1, esem2, zsem):
        cid = lax.axis_index("c")
        sid = lax.axis_index("s")
        wid = cid * NS + sid
        row_base = pl.multiple_of(sid * RPS, 8)
        last_base = (NS - 1) * RPS + RPS
        cbase = wid * NF

        ibs = (ib0, ib1, ib2)
        ebs = (eb0, eb1, eb2)
        rbs = (rb0, rb1, rb2)
        gsems = (gsem0, gsem1, gsem2)
        isems = (isem0, isem1, isem2)
        esems = (esem0, esem1, esem2)

        pltpu.async_copy(zeros_hbm.at[pl.ds(0, RPS)],
                         acc_sh.at[pl.ds(row_base, RPS)], zsem)

        @pl.when(sid == NS - 1)
        def _():
            if RLAST != RPS:
                pltpu.async_copy(
                    zeros_hbm.at[pl.ds(RPS, RLAST - RPS)],
                    acc_sh.at[pl.ds(last_base, RLAST - RPS)], zsem)

        def load_idx(s, j):
            off = (cbase + s) * W
            pltpu.async_copy(ei_hbm.at[:, pl.ds(off, W)], ibs[j], isems[j])
            pltpu.async_copy(et_hbm.at[pl.ds(off, W)], ebs[j], esems[j])

        def wait_idx(s, j):
            off = (cbase + s) * W
            pltpu.make_async_copy(ei_hbm.at[:, pl.ds(off, W)], ibs[j],
                                  isems[j]).wait()
            pltpu.make_async_copy(et_hbm.at[pl.ds(off, W)], ebs[j],
                                  esems[j]).wait()
            for k in range(W // 16):
                sl = pl.ds(16 * k, 16)
                ibs[j][0, sl] = ebs[j][sl] * N + ibs[j][0, sl]

        def gather(j):
            return pltpu.async_copy(hrw_hbm.at[ibs[j].at[0]],
                                    rbs[j], gsems[j])

        def wait_gather(j):
            pltpu.make_async_copy(hrw_hbm.at[ibs[j].at[0]],
                                  rbs[j], gsems[j]).wait()

        def scat(j):
            pltpu.sync_copy(rbs[j], acc_sh.at[ibs[j].at[1]], add=True)

        load_idx(0, 0)
        load_idx(1, 1)
        wait_idx(0, 0)
        wait_idx(1, 1)
        gather(0)
        gather(1)
        load_idx(2, 2)

        pltpu.make_async_copy(zeros_hbm.at[pl.ds(0, RPS)],
                              acc_sh.at[pl.ds(row_base, RPS)], zsem).wait()

        @pl.when(sid == NS - 1)
        def _():
            if RLAST != RPS:
                pltpu.make_async_copy(
                    zeros_hbm.at[pl.ds(RPS, RLAST - RPS)],
                    acc_sh.at[pl.ds(last_base, RLAST - RPS)], zsem).wait()

        plsc.subcore_barrier()

        @pl.loop(0, (NF - 3) // 3)
        def _(g):
            for j in range(3):
                s = g * 3 + j
                wait_gather(j)
                wait_idx(s + 2, (j + 2) % 3)
                gather((j + 2) % 3)
                scat(j)
                load_idx(s + 3, j)

        wait_gather(0)
        wait_idx(NF - 1, 2)
        gather(2)
        scat(0)

        xoff = (NW * NF + wid) * W

        @pl.when(wid < NEXTRA)
        def _():
            pltpu.async_copy(ei_hbm.at[:, pl.ds(xoff, W)], ib0, isem0)
            pltpu.async_copy(et_hbm.at[pl.ds(xoff, W)], eb0, esem0)

        wait_gather(1)
        scat(1)

        @pl.when(wid < NEXTRA)
        def _():
            pltpu.make_async_copy(ei_hbm.at[:, pl.ds(xoff, W)], ib0,
                                  isem0).wait()
            pltpu.make_async_copy(et_hbm.at[pl.ds(xoff, W)], eb0,
                                  esem0).wait()
            for k in range(W // 16):
                sl = pl.ds(16 * k, 16)
                ib0[0, sl] = eb0[sl] * N + ib0[0, sl]
            pltpu.async_copy(hrw_hbm.at[ib0.at[0]], rb0, gsem0)

        wait_gather(2)
        scat(2)

        @pl.when(wid < NEXTRA)
        def _():
            pltpu.make_async_copy(hrw_hbm.at[ib0.at[0]], rb0, gsem0).wait()
            pltpu.sync_copy(rb0, acc_sh.at[ib0.at[1]], add=True)

        plsc.subcore_barrier()

        @pl.when(sid != NS - 1)
        def _():
            pltpu.sync_copy(acc_sh.at[pl.ds(row_base, RPS)],
                            out_hbm.at[cid, pl.ds(row_base, RPS)])

        @pl.when(sid == NS - 1)
        def _():
            pltpu.sync_copy(acc_sh.at[pl.ds((NS - 1) * RPS, RLAST)],
                            out_hbm.at[cid, pl.ds((NS - 1) * RPS, RLAST)])

    return sc_agg


def kernel(x, edge_index, edge_type, num_edge_types,
           w0, eps0, m0w1, m0b1, m0w2, m0b2,
           w1, eps1, m1w1, m1b1, m1w2, m1b2):
    N, C = x.shape
    E = edge_type.shape[0]
    R = w0.shape[0]

    sc_agg = _make_sc_agg(N, C, E)
    mlp_fused = _make_mlp(N, C, R, fuse_premult=True)
    mlp_final = _make_mlp(N, C, R, fuse_premult=False)

    NS = 16
    rps = (N // NS) // 8 * 8
    rlast = N - (NS - 1) * rps
    zeros = jnp.zeros((rlast, C), jnp.float32)
    hrw0 = _make_premult(N, C, R)(x, w0)
    part0 = sc_agg(hrw0.reshape(R * N, C), edge_index, edge_type, zeros)
    h1, hrw1 = mlp_fused(eps0.reshape(1, 1), x, part0,
                         m0w1, m0b1.reshape(1, C), m0w2, m0b2.reshape(1, C),
                         w1)
    part1 = sc_agg(hrw1.reshape(R * N, C), edge_index, edge_type, zeros)
    out = mlp_final(eps1.reshape(1, 1), h1, part1,
                    m1w1, m1b1.reshape(1, C), m1w2, m1b2.reshape(1, C))
    return out
